# baseline (speedup 1.0000x reference)
import jax
import jax.numpy as jnp
from jax.experimental import pallas as pl

_NUM_EXPERTS = 64
_TOP_K = 8
_BLOCK_T = 2048


def _router_body(x_ref, w_ref, wout_ref, iout_ref):
    logits = jax.lax.dot_general(
        w_ref[...],
        x_ref[...],
        dimension_numbers=(((0,), (1,)), ((), ())),
        preferred_element_type=jnp.float32,
    )
    bt = logits.shape[1]
    m = jnp.max(logits, axis=0, keepdims=True)
    e = jnp.exp(logits - m)
    eidx = jax.lax.broadcasted_iota(jnp.int32, (_NUM_EXPERTS, bt), 0)

    work = e
    vals = []
    ids = []
    for _ in range(_TOP_K):
        cur = jnp.max(work, axis=0, keepdims=True)
        idx = jnp.min(
            jnp.where(work == cur, eidx, _NUM_EXPERTS), axis=0, keepdims=True
        )
        vals.append(cur)
        ids.append(idx)
        work = jnp.where(eidx == idx, -1.0, work)

    v = jnp.concatenate(vals, axis=0)
    i = jnp.concatenate(ids, axis=0)
    w = v / jnp.sum(v, axis=0, keepdims=True)
    wout_ref[...] = w.T
    iout_ref[...] = jax.lax.bitcast_convert_type(
        jax.lax.bitcast_convert_type(i, jnp.float32).T, jnp.int32
    )


@jax.jit
def kernel(x, W_router):
    tokens = x.shape[0]
    grid = (tokens // _BLOCK_T,)
    weights_t, ids_t = pl.pallas_call(
        _router_body,
        grid=grid,
        in_specs=[
            pl.BlockSpec((_BLOCK_T, x.shape[1]), lambda t: (t, 0)),
            pl.BlockSpec((x.shape[1], _NUM_EXPERTS), lambda t: (0, 0)),
        ],
        out_specs=[
            pl.BlockSpec((_BLOCK_T, _TOP_K), lambda t: (t, 0)),
            pl.BlockSpec((_BLOCK_T, _TOP_K), lambda t: (t, 0)),
        ],
        out_shape=[
            jax.ShapeDtypeStruct((tokens, _TOP_K), jnp.float32),
            jax.ShapeDtypeStruct((tokens, _TOP_K), jnp.int32),
        ],
    )(x, W_router)
    return weights_t, ids_t


# speedup vs baseline: 1.2642x; 1.2642x over previous
import jax
import jax.numpy as jnp
from jax.experimental import pallas as pl

_NUM_EXPERTS = 64
_TOP_K = 8
_BLOCK_T = 2048


def _router_body(x_ref, w_ref, wout_ref, iout_ref):
    logits = jax.lax.dot_general(
        w_ref[...],
        x_ref[...],
        dimension_numbers=(((0,), (1,)), ((), ())),
        preferred_element_type=jnp.float32,
    )
    bt = logits.shape[1]
    m = jnp.max(logits, axis=0, keepdims=True)
    e = jnp.exp(logits - m)
    eidx = jax.lax.broadcasted_iota(jnp.int32, (_NUM_EXPERTS, bt), 0)

    work = e
    vals = []
    ids = []
    for _ in range(_TOP_K):
        cur = jnp.max(work, axis=0, keepdims=True)
        idx = jnp.min(
            jnp.where(work == cur, eidx, _NUM_EXPERTS), axis=0, keepdims=True
        )
        vals.append(cur)
        ids.append(idx)
        work = jnp.where(eidx == idx, -1.0, work)

    v = jnp.concatenate(vals, axis=0)
    i = jnp.concatenate(ids, axis=0)
    wout_ref[...] = v / jnp.sum(v, axis=0, keepdims=True)
    iout_ref[...] = i


@jax.jit
def kernel(x, W_router):
    tokens = x.shape[0]
    grid = (tokens // _BLOCK_T,)
    weights_t, ids_t = pl.pallas_call(
        _router_body,
        grid=grid,
        in_specs=[
            pl.BlockSpec((_BLOCK_T, x.shape[1]), lambda t: (t, 0)),
            pl.BlockSpec((x.shape[1], _NUM_EXPERTS), lambda t: (0, 0)),
        ],
        out_specs=[
            pl.BlockSpec((_TOP_K, _BLOCK_T), lambda t: (0, t)),
            pl.BlockSpec((_TOP_K, _BLOCK_T), lambda t: (0, t)),
        ],
        out_shape=[
            jax.ShapeDtypeStruct((_TOP_K, tokens), jnp.float32),
            jax.ShapeDtypeStruct((_TOP_K, tokens), jnp.int32),
        ],
    )(x, W_router)
    return weights_t.T, ids_t.T
